# Initial kernel scaffold; baseline (speedup 1.0000x reference)
import jax
import jax.numpy as jnp
from jax.experimental import pallas as pl


def kernel(hn, he, edge_index, edge_vec, emb, norm, W_fc1, W_fc2, W2_fc1, W2_fc2, W_lin1, W_lin2):
    raise NotImplementedError("write your pallas kernel here")


# speedup vs baseline: 4.5979x; 4.5979x over previous
import functools

import numpy as np
import jax
import jax.numpy as jnp
from jax import lax
from jax.experimental import pallas as pl
from jax.experimental.pallas import tpu as pltpu
from jax.experimental.pallas import tpu_sc as plsc

N_NODES = 10000
E_EDGES = 160000
D = 16
EB = 10
C_RELU = float(np.sqrt(2.0))
C_TANH = 1.5927

NSLAB = 128 // D
RS = E_EDGES // NSLAB

NC = 2
NS = 16
NW = NC * NS
CH = 128
NROW = E_EDGES // CH
NROWP = 1256
NCH = 40
NRC = 1000
NS_OUT = N_NODES // NRC


@functools.cache
def _sc_gather_kernel():
    mesh = plsc.VectorSubcoreMesh(core_axis_name="c", subcore_axis_name="s",
                                  num_cores=NC, num_subcores=NS)
    return functools.partial(
        pl.kernel,
        out_type=[jax.ShapeDtypeStruct((E_EDGES, D), jnp.float32),
                  jax.ShapeDtypeStruct((E_EDGES, D), jnp.float32)],
        mesh=mesh,
        scratch_types=[
            pltpu.VMEM((NCH, CH), jnp.int32),
            pltpu.VMEM((NCH, CH), jnp.int32),
            pltpu.VMEM((NCH, CH, D), jnp.float32),
            pltpu.SemaphoreType.DMA,
            pltpu.SemaphoreType.DMA,
        ],
        compiler_params=pltpu.CompilerParams(use_tc_tiling_on_sc=False),
    )(_sc_gather_body)


def _gather_phase(hn_hbm, idx, rows, out_hbm, row0, nch, sem_g, sem_w):
    def fire(j, carry):
        pltpu.async_copy(hn_hbm.at[idx.at[j]], rows.at[j], sem_g)
        return carry

    lax.fori_loop(0, nch, fire, 0)

    def drain(j, carry):
        pltpu.make_async_copy(hn_hbm.at[idx.at[j]], rows.at[j], sem_g).wait()
        return carry

    lax.fori_loop(0, nch, drain, 0)

    def fire_w(j, carry):
        pltpu.async_copy(rows.at[j], out_hbm.at[pl.ds((row0 + j) * CH, CH)],
                         sem_w)
        return carry

    lax.fori_loop(0, nch, fire_w, 0)

    def dwait(j, carry):
        pltpu.make_async_copy(
            rows.at[j], out_hbm.at[pl.ds((row0 + j) * CH, CH)], sem_w).wait()
        return carry

    lax.fori_loop(0, nch, dwait, 0)


def _sc_gather_body(hn_hbm, src_hbm, dst_hbm, osrc_hbm, odst_hbm,
                    sidx, didx, rows, sem_g, sem_w):
    wid = lax.axis_index("s") * NC + lax.axis_index("c")
    row0 = wid * NCH
    nch = jnp.where(wid == NW - 1, NROW - (NW - 1) * NCH, NCH)

    @pl.when(wid < NW - 1)
    def _():
        pltpu.sync_copy(src_hbm.at[pl.ds(row0, NCH)], sidx)
        pltpu.sync_copy(dst_hbm.at[pl.ds(row0, NCH)], didx)

    @pl.when(wid == NW - 1)
    def _():
        pltpu.sync_copy(src_hbm.at[pl.ds(row0, 16)], sidx.at[pl.ds(0, 16)])
        pltpu.sync_copy(dst_hbm.at[pl.ds(row0, 16)], didx.at[pl.ds(0, 16)])

    _gather_phase(hn_hbm, sidx, rows, osrc_hbm, row0, nch, sem_g, sem_w)
    _gather_phase(hn_hbm, didx, rows, odst_hbm, row0, nch, sem_g, sem_w)


@functools.cache
def _sc_scatter_kernel():
    mesh = plsc.VectorSubcoreMesh(core_axis_name="c", subcore_axis_name="s",
                                  num_cores=NC, num_subcores=NS)
    return functools.partial(
        pl.kernel,
        out_type=jax.ShapeDtypeStruct((NC, N_NODES, D), jnp.float32),
        mesh=mesh,
        scratch_types=[
            pltpu.VMEM((NCH, CH), jnp.int32),
            pltpu.VMEM((NCH, CH, D), jnp.float32),
            pltpu.VMEM((NRC, D), jnp.float32),
            pltpu.VMEM_SHARED((N_NODES, D), jnp.float32),
            pltpu.SemaphoreType.DMA,
        ],
        compiler_params=pltpu.CompilerParams(use_tc_tiling_on_sc=False),
    )(_sc_scatter_body)


def _sc_scatter_body(contrib_hbm, dst_hbm, out_hbm,
                     idx_v, rows, tbuf, acc_sh, sem_l):
    c = lax.axis_index("c")
    s = lax.axis_index("s")
    wid = s * NC + c
    row0 = wid * NCH
    nch = jnp.where(wid == NW - 1, NROW - (NW - 1) * NCH, NCH)

    def zb(i, carry):
        tbuf[i, :] = jnp.zeros((D,), jnp.float32)
        return carry

    lax.fori_loop(0, NRC, zb, 0)

    @pl.when(s < NS_OUT)
    def _():
        pltpu.sync_copy(tbuf, acc_sh.at[pl.ds(s * NRC, NRC)])

    @pl.when(wid < NW - 1)
    def _():
        pltpu.sync_copy(dst_hbm.at[pl.ds(row0, NCH)], idx_v)

    @pl.when(wid == NW - 1)
    def _():
        pltpu.sync_copy(dst_hbm.at[pl.ds(row0, 16)], idx_v.at[pl.ds(0, 16)])

    def fire(j, carry):
        pltpu.async_copy(contrib_hbm.at[pl.ds((row0 + j) * CH, CH)],
                         rows.at[j], sem_l)
        return carry

    lax.fori_loop(0, nch, fire, 0)

    def drain(j, carry):
        pltpu.make_async_copy(contrib_hbm.at[pl.ds((row0 + j) * CH, CH)],
                              rows.at[j], sem_l).wait()
        return carry

    lax.fori_loop(0, nch, drain, 0)
    plsc.subcore_barrier()

    def body(j, carry):
        pltpu.sync_copy(rows.at[j], acc_sh.at[idx_v.at[j]], add=True)
        return carry

    lax.fori_loop(0, nch, body, 0)
    plsc.subcore_barrier()

    @pl.when(s < NS_OUT)
    def _():
        pltpu.sync_copy(acc_sh.at[pl.ds(s * NRC, NRC)], tbuf)
        pltpu.sync_copy(tbuf, out_hbm.at[c].at[pl.ds(s * NRC, NRC)])


RB = 1000
EGRID = RS // RB


def _fold16(m):
    a = m[:, :128] + m[:, 128:]
    a = a[:, :64] + a[:, 64:]
    a = a[:, :32] + a[:, 32:]
    return a[:, :16] + a[:, 16:]


def _edge_body(he_ref, hsp_ref, hdp_ref, emb_ref, normp_ref,
               w12r_ref, w3_ref, w4_ref, henew_ref, contribp_ref):
    w12r = w12r_ref[...]
    w3 = w3_ref[...]
    w4 = w4_ref[...]
    hsp = hsp_ref[...]
    hdp = hdp_ref[...]
    normp = normp_ref[...]
    contrib_parts = []
    for s in range(NSLAB):
        he = he_ref[s]
        hs = hsp[:, s * D:(s + 1) * D]
        hd = hdp[:, s * D:(s + 1) * D]
        x_cat = jnp.concatenate([he, hs, hd], axis=1)
        hr = C_RELU * jax.nn.relu(emb_ref[s] @ w12r)
        t2 = x_cat @ w3
        tp = _fold16(t2 * hr[:, :256])
        tmp = C_TANH * jnp.tanh(tp)
        t3 = tmp @ w4
        he_new = he + _fold16(t3 * hr[:, 256:])
        henew_ref[s] = he_new
        contrib_parts.append(he_new * normp[:, s * D:(s + 1) * D])
    contribp_ref[...] = jnp.concatenate(contrib_parts, axis=1)


def _edge_call(he4, hsp, hdp, emb4, normp, w12r, w3c, w4c):
    return pl.pallas_call(
        _edge_body,
        grid=(EGRID,),
        in_specs=[
            pl.BlockSpec((NSLAB, RB, D), lambda i: (0, i, 0)),
            pl.BlockSpec((RB, 128), lambda i: (i, 0)),
            pl.BlockSpec((RB, 128), lambda i: (i, 0)),
            pl.BlockSpec((NSLAB, RB, EB), lambda i: (0, i, 0)),
            pl.BlockSpec((RB, 128), lambda i: (i, 0)),
            pl.BlockSpec((EB, 2 * D * D), lambda i: (0, 0)),
            pl.BlockSpec((3 * D, D * D), lambda i: (0, 0)),
            pl.BlockSpec((D, D * D), lambda i: (0, 0)),
        ],
        out_specs=[
            pl.BlockSpec((NSLAB, RB, D), lambda i: (0, i, 0)),
            pl.BlockSpec((RB, 128), lambda i: (i, 0)),
        ],
        out_shape=[
            jax.ShapeDtypeStruct((NSLAB, RS, D), jnp.float32),
            jax.ShapeDtypeStruct((RS, 128), jnp.float32),
        ],
        compiler_params=pltpu.CompilerParams(
            dimension_semantics=("arbitrary",)),
    )(he4, hsp, hdp, emb4, normp, w12r, w3c, w4c)


def _node_body(hn_ref, p_ref, wl1a_ref, wl1b_ref, wl2_ref, out_ref):
    hn = hn_ref[...]
    nf = p_ref[0] + p_ref[1]
    z = C_TANH * jnp.tanh(hn @ wl1a_ref[...] + nf @ wl1b_ref[...])
    out_ref[...] = hn + z @ wl2_ref[...]


def _node_call(hn, partials, wl1a, wl1b, wl2, blk, grid):
    return pl.pallas_call(
        _node_body,
        grid=(grid,),
        in_specs=[
            pl.BlockSpec((blk, D), lambda i: (i, 0)),
            pl.BlockSpec((NC, blk, D), lambda i: (0, i, 0)),
            pl.BlockSpec((D, D), lambda i: (0, 0)),
            pl.BlockSpec((D, D), lambda i: (0, 0)),
            pl.BlockSpec((D, D), lambda i: (0, 0)),
        ],
        out_specs=pl.BlockSpec((blk, D), lambda i: (i, 0)),
        out_shape=jax.ShapeDtypeStruct((N_NODES, D), jnp.float32),
        compiler_params=pltpu.CompilerParams(
            dimension_semantics=("arbitrary",)),
    )(hn, partials, wl1a, wl1b, wl2)


def _prep_weights(W_fc1, W_fc2, W2_fc1, W2_fc2, W_lin1, W_lin2):
    s_eb = 1.0 / np.sqrt(float(EB))
    w1r = jnp.repeat(W_fc1 * s_eb, D, axis=1)
    w2r = jnp.repeat(W2_fc1 * s_eb, D, axis=1)
    w12r = jnp.concatenate([w1r, w2r], axis=1)
    w3c = (W_fc2.reshape(D, 3 * D, D).transpose(1, 0, 2).reshape(3 * D, D * D)
           * (1.0 / (4.0 * np.sqrt(float(3 * D)))))
    w4c = (W2_fc2.reshape(D, D, D).transpose(1, 0, 2).reshape(D, D * D)
           * (1.0 / 16.0))
    s32 = 1.0 / np.sqrt(float(2 * D))
    wl1a = W_lin1[:D] * s32
    wl1b = W_lin1[D:] * s32
    wl2 = W_lin2 * 0.25
    return w12r, w3c, w4c, wl1a, wl1b, wl2


def _to_q(x):
    return x.reshape(NSLAB, RS).transpose(1, 0).reshape(E_EDGES)


def kernel(hn, he, edge_index, edge_vec, emb, norm,
           W_fc1, W_fc2, W2_fc1, W2_fc2, W_lin1, W_lin2):
    del edge_vec
    pad_idx = (NROWP - NROW) * CH
    src2 = jnp.pad(_to_q(edge_index[0].astype(jnp.int32)),
                   (0, pad_idx)).reshape(NROWP, CH)
    dst2 = jnp.pad(_to_q(edge_index[1].astype(jnp.int32)),
                   (0, pad_idx)).reshape(NROWP, CH)

    hn_src, hn_dst = _sc_gather_kernel()(hn, src2, dst2)
    hsp = hn_src.reshape(RS, 128)
    hdp = hn_dst.reshape(RS, 128)

    he4 = he.reshape(NSLAB, RS, D)
    emb4 = emb.reshape(NSLAB, RS, EB)
    normp = jnp.broadcast_to(
        norm.reshape(NSLAB, RS).transpose(1, 0)[:, :, None],
        (RS, NSLAB, D)).reshape(RS, 128)

    w12r, w3c, w4c, wl1a, wl1b, wl2 = _prep_weights(
        W_fc1, W_fc2, W2_fc1, W2_fc2, W_lin1, W_lin2)

    henew4, contribp = _edge_call(he4, hsp, hdp, emb4, normp,
                                  w12r, w3c, w4c)
    he_new = henew4.reshape(E_EDGES, D)

    partials = _sc_scatter_kernel()(contribp.reshape(E_EDGES, D), dst2)

    hn_new = _node_call(hn, partials, wl1a, wl1b, wl2, blk=2000, grid=5)
    return hn_new, he_new


# speedup vs baseline: 4.7951x; 1.0429x over previous
import functools

import numpy as np
import jax
import jax.numpy as jnp
from jax import lax
from jax.experimental import pallas as pl
from jax.experimental.pallas import tpu as pltpu
from jax.experimental.pallas import tpu_sc as plsc

N_NODES = 10000
E_EDGES = 160000
D = 16
EB = 10
C_RELU = float(np.sqrt(2.0))
C_TANH = 1.5927

NSLAB = 128 // D
RS = E_EDGES // NSLAB

NC = 2
NS = 16
NW = NC * NS
CH = 128
NROW = E_EDGES // CH
NROWP = 1256
NCH = 40
NRC = 1000
NS_OUT = N_NODES // NRC


@functools.cache
def _sc_gather_kernel():
    mesh = plsc.VectorSubcoreMesh(core_axis_name="c", subcore_axis_name="s",
                                  num_cores=NC, num_subcores=NS)
    return functools.partial(
        pl.kernel,
        out_type=[jax.ShapeDtypeStruct((E_EDGES, D), jnp.float32),
                  jax.ShapeDtypeStruct((E_EDGES, D), jnp.float32)],
        mesh=mesh,
        scratch_types=[
            pltpu.VMEM((NCH, CH), jnp.int32),
            pltpu.VMEM((NCH, CH), jnp.int32),
            pltpu.VMEM((NCH, CH, D), jnp.float32),
            pltpu.SemaphoreType.DMA,
            pltpu.SemaphoreType.DMA,
        ],
        compiler_params=pltpu.CompilerParams(use_tc_tiling_on_sc=False),
    )(_sc_gather_body)


def _gather_phase(hn_hbm, idx, rows, out_hbm, row0, nch, sem_g, sem_w):
    def fire(j, carry):
        pltpu.async_copy(hn_hbm.at[idx.at[j]], rows.at[j], sem_g)
        return carry

    lax.fori_loop(0, nch, fire, 0)

    def drain(j, carry):
        pltpu.make_async_copy(hn_hbm.at[idx.at[j]], rows.at[j], sem_g).wait()
        return carry

    lax.fori_loop(0, nch, drain, 0)

    def fire_w(j, carry):
        pltpu.async_copy(rows.at[j], out_hbm.at[pl.ds((row0 + j) * CH, CH)],
                         sem_w)
        return carry

    lax.fori_loop(0, nch, fire_w, 0)

    def dwait(j, carry):
        pltpu.make_async_copy(
            rows.at[j], out_hbm.at[pl.ds((row0 + j) * CH, CH)], sem_w).wait()
        return carry

    lax.fori_loop(0, nch, dwait, 0)


def _sc_gather_body(hn_hbm, src_hbm, dst_hbm, osrc_hbm, odst_hbm,
                    sidx, didx, rows, sem_g, sem_w):
    wid = lax.axis_index("s") * NC + lax.axis_index("c")
    row0 = wid * NCH
    nch = jnp.where(wid == NW - 1, NROW - (NW - 1) * NCH, NCH)

    @pl.when(wid < NW - 1)
    def _():
        pltpu.sync_copy(src_hbm.at[pl.ds(row0, NCH)], sidx)
        pltpu.sync_copy(dst_hbm.at[pl.ds(row0, NCH)], didx)

    @pl.when(wid == NW - 1)
    def _():
        pltpu.sync_copy(src_hbm.at[pl.ds(row0, 16)], sidx.at[pl.ds(0, 16)])
        pltpu.sync_copy(dst_hbm.at[pl.ds(row0, 16)], didx.at[pl.ds(0, 16)])

    _gather_phase(hn_hbm, sidx, rows, osrc_hbm, row0, nch, sem_g, sem_w)
    _gather_phase(hn_hbm, didx, rows, odst_hbm, row0, nch, sem_g, sem_w)


@functools.cache
def _sc_scatter_kernel():
    mesh = plsc.VectorSubcoreMesh(core_axis_name="c", subcore_axis_name="s",
                                  num_cores=NC, num_subcores=NS)
    return functools.partial(
        pl.kernel,
        out_type=jax.ShapeDtypeStruct((NC, N_NODES, D), jnp.float32),
        mesh=mesh,
        scratch_types=[
            pltpu.VMEM((NCH, CH), jnp.int32),
            pltpu.VMEM((NCH, CH, D), jnp.float32),
            pltpu.VMEM((NRC, D), jnp.float32),
            pltpu.VMEM_SHARED((N_NODES, D), jnp.float32),
            pltpu.SemaphoreType.DMA,
        ],
        compiler_params=pltpu.CompilerParams(use_tc_tiling_on_sc=False),
    )(_sc_scatter_body)


def _sc_scatter_body(contrib_hbm, dst_hbm, out_hbm,
                     idx_v, rows, tbuf, acc_sh, sem_l):
    c = lax.axis_index("c")
    s = lax.axis_index("s")
    wid = s * NC + c
    row0 = wid * NCH
    nch = jnp.where(wid == NW - 1, NROW - (NW - 1) * NCH, NCH)

    def zb(i, carry):
        tbuf[i, :] = jnp.zeros((D,), jnp.float32)
        return carry

    lax.fori_loop(0, NRC, zb, 0)

    @pl.when(s < NS_OUT)
    def _():
        pltpu.sync_copy(tbuf, acc_sh.at[pl.ds(s * NRC, NRC)])

    @pl.when(wid < NW - 1)
    def _():
        pltpu.sync_copy(dst_hbm.at[pl.ds(row0, NCH)], idx_v)

    @pl.when(wid == NW - 1)
    def _():
        pltpu.sync_copy(dst_hbm.at[pl.ds(row0, 16)], idx_v.at[pl.ds(0, 16)])

    def fire(j, carry):
        pltpu.async_copy(contrib_hbm.at[pl.ds((row0 + j) * CH, CH)],
                         rows.at[j], sem_l)
        return carry

    lax.fori_loop(0, nch, fire, 0)

    def drain(j, carry):
        pltpu.make_async_copy(contrib_hbm.at[pl.ds((row0 + j) * CH, CH)],
                              rows.at[j], sem_l).wait()
        return carry

    lax.fori_loop(0, nch, drain, 0)
    plsc.subcore_barrier()

    def body(j, carry):
        pltpu.sync_copy(rows.at[j], acc_sh.at[idx_v.at[j]], add=True)
        return carry

    lax.fori_loop(0, nch, body, 0)
    plsc.subcore_barrier()

    @pl.when(s < NS_OUT)
    def _():
        pltpu.sync_copy(acc_sh.at[pl.ds(s * NRC, NRC)], tbuf)
        pltpu.sync_copy(tbuf, out_hbm.at[c].at[pl.ds(s * NRC, NRC)])


RB = 1000
EGRID = RS // RB


def _fold16(m):
    a = m[:, :128] + m[:, 128:]
    a = a[:, :64] + a[:, 64:]
    a = a[:, :32] + a[:, 32:]
    return a[:, :16] + a[:, 16:]


def _edge_body(he_ref, hsp_ref, hdp_ref, emb_ref, normp_ref,
               w12r_ref, w3_ref, w4_ref, henew_ref, contribp_ref):
    hsp = hsp_ref[...]
    hdp = hdp_ref[...]
    he = he_ref[...].reshape(NSLAB * RB, D)
    hs = jnp.concatenate([hsp[:, s * D:(s + 1) * D] for s in range(NSLAB)],
                         axis=0)
    hd = jnp.concatenate([hdp[:, s * D:(s + 1) * D] for s in range(NSLAB)],
                         axis=0)
    emb = emb_ref[...].reshape(NSLAB * RB, EB)
    x_cat = jnp.concatenate([he, hs, hd], axis=1)
    hr = C_RELU * jax.nn.relu(emb @ w12r_ref[...])
    t2 = x_cat @ w3_ref[...]
    tp = _fold16(t2 * hr[:, :256])
    tmp = C_TANH * jnp.tanh(tp)
    t3 = tmp @ w4_ref[...]
    he_new = he + _fold16(t3 * hr[:, 256:])
    henew_ref[...] = he_new.reshape(NSLAB, RB, D)
    normp = normp_ref[...]
    contribp_ref[...] = jnp.concatenate(
        [he_new[s * RB:(s + 1) * RB] * normp[:, s * D:(s + 1) * D]
         for s in range(NSLAB)], axis=1)


def _edge_call(he4, hsp, hdp, emb4, normp, w12r, w3c, w4c):
    return pl.pallas_call(
        _edge_body,
        grid=(EGRID,),
        in_specs=[
            pl.BlockSpec((NSLAB, RB, D), lambda i: (0, i, 0)),
            pl.BlockSpec((RB, 128), lambda i: (i, 0)),
            pl.BlockSpec((RB, 128), lambda i: (i, 0)),
            pl.BlockSpec((NSLAB, RB, EB), lambda i: (0, i, 0)),
            pl.BlockSpec((RB, 128), lambda i: (i, 0)),
            pl.BlockSpec((EB, 2 * D * D), lambda i: (0, 0)),
            pl.BlockSpec((3 * D, D * D), lambda i: (0, 0)),
            pl.BlockSpec((D, D * D), lambda i: (0, 0)),
        ],
        out_specs=[
            pl.BlockSpec((NSLAB, RB, D), lambda i: (0, i, 0)),
            pl.BlockSpec((RB, 128), lambda i: (i, 0)),
        ],
        out_shape=[
            jax.ShapeDtypeStruct((NSLAB, RS, D), jnp.float32),
            jax.ShapeDtypeStruct((RS, 128), jnp.float32),
        ],
        compiler_params=pltpu.CompilerParams(
            dimension_semantics=("arbitrary",)),
    )(he4, hsp, hdp, emb4, normp, w12r, w3c, w4c)


def _node_body(hn_ref, p_ref, wl1a_ref, wl1b_ref, wl2_ref, out_ref):
    hn = hn_ref[...]
    nf = p_ref[0] + p_ref[1]
    z = C_TANH * jnp.tanh(hn @ wl1a_ref[...] + nf @ wl1b_ref[...])
    out_ref[...] = hn + z @ wl2_ref[...]


def _node_call(hn, partials, wl1a, wl1b, wl2, blk, grid):
    return pl.pallas_call(
        _node_body,
        grid=(grid,),
        in_specs=[
            pl.BlockSpec((blk, D), lambda i: (i, 0)),
            pl.BlockSpec((NC, blk, D), lambda i: (0, i, 0)),
            pl.BlockSpec((D, D), lambda i: (0, 0)),
            pl.BlockSpec((D, D), lambda i: (0, 0)),
            pl.BlockSpec((D, D), lambda i: (0, 0)),
        ],
        out_specs=pl.BlockSpec((blk, D), lambda i: (i, 0)),
        out_shape=jax.ShapeDtypeStruct((N_NODES, D), jnp.float32),
        compiler_params=pltpu.CompilerParams(
            dimension_semantics=("arbitrary",)),
    )(hn, partials, wl1a, wl1b, wl2)


def _prep_weights(W_fc1, W_fc2, W2_fc1, W2_fc2, W_lin1, W_lin2):
    s_eb = 1.0 / np.sqrt(float(EB))
    w1r = jnp.repeat(W_fc1 * s_eb, D, axis=1)
    w2r = jnp.repeat(W2_fc1 * s_eb, D, axis=1)
    w12r = jnp.concatenate([w1r, w2r], axis=1)
    w3c = (W_fc2.reshape(D, 3 * D, D).transpose(1, 0, 2).reshape(3 * D, D * D)
           * (1.0 / (4.0 * np.sqrt(float(3 * D)))))
    w4c = (W2_fc2.reshape(D, D, D).transpose(1, 0, 2).reshape(D, D * D)
           * (1.0 / 16.0))
    s32 = 1.0 / np.sqrt(float(2 * D))
    wl1a = W_lin1[:D] * s32
    wl1b = W_lin1[D:] * s32
    wl2 = W_lin2 * 0.25
    return w12r, w3c, w4c, wl1a, wl1b, wl2


def _to_q(x):
    return x.reshape(NSLAB, RS).transpose(1, 0).reshape(E_EDGES)


def kernel(hn, he, edge_index, edge_vec, emb, norm,
           W_fc1, W_fc2, W2_fc1, W2_fc2, W_lin1, W_lin2):
    del edge_vec
    pad_idx = (NROWP - NROW) * CH
    src2 = jnp.pad(_to_q(edge_index[0].astype(jnp.int32)),
                   (0, pad_idx)).reshape(NROWP, CH)
    dst2 = jnp.pad(_to_q(edge_index[1].astype(jnp.int32)),
                   (0, pad_idx)).reshape(NROWP, CH)

    hn_src, hn_dst = _sc_gather_kernel()(hn, src2, dst2)
    hsp = hn_src.reshape(RS, 128)
    hdp = hn_dst.reshape(RS, 128)

    he4 = he.reshape(NSLAB, RS, D)
    emb4 = emb.reshape(NSLAB, RS, EB)
    normp = jnp.broadcast_to(
        norm.reshape(NSLAB, RS).transpose(1, 0)[:, :, None],
        (RS, NSLAB, D)).reshape(RS, 128)

    w12r, w3c, w4c, wl1a, wl1b, wl2 = _prep_weights(
        W_fc1, W_fc2, W2_fc1, W2_fc2, W_lin1, W_lin2)

    henew4, contribp = _edge_call(he4, hsp, hdp, emb4, normp,
                                  w12r, w3c, w4c)
    he_new = henew4.reshape(E_EDGES, D)

    partials = _sc_scatter_kernel()(contribp.reshape(E_EDGES, D), dst2)

    hn_new = _node_call(hn, partials, wl1a, wl1b, wl2, blk=2000, grid=5)
    return hn_new, he_new
